# baseline (speedup 1.0000x reference)
import functools

import jax
import jax.numpy as jnp
from jax import lax
from jax.experimental import pallas as pl
from jax.experimental.pallas import tpu as pltpu
from jax.experimental.pallas import tpu_sc as plsc

_K = 32
_D = 768
_BLK = 256
_NTOK = 16384
_HALVES = 2
_NW = 32
_RPW = _NTOK // _HALVES // _NW
_CH = 8
_IL = 8


def _enc_body(x_ref, we_ref, pb_ref, lb_ref, o_ref):
    xc = x_ref[...] - pb_ref[...]
    acts = jnp.dot(xc, we_ref[...], preferred_element_type=jnp.float32)
    o_ref[...] = acts + lb_ref[...]


def _dec_body(a_ref, t_ref, wd_ref, pb_ref, o_ref):
    acts = a_ref[...]
    keep = acts >= t_ref[...]
    acts_topk = jnp.where(keep, jnp.maximum(acts, 0.0), 0.0)
    out = jnp.dot(acts_topk.astype(jnp.bfloat16), wd_ref[...],
                  preferred_element_type=jnp.float32)
    o_ref[...] = out + pb_ref[...]


def _rev(x):
    return lax.rev(x, (0,))


def _sorted32(a, b):
    rb = _rev(b)
    return jnp.sort(jnp.minimum(a, rb)), jnp.sort(jnp.maximum(a, rb))


def _top32_merge(t0, t1, w0, w1):
    p0 = jnp.maximum(t0, _rev(w1))
    p1 = jnp.maximum(t1, _rev(w0))
    return jnp.sort(jnp.minimum(p0, p1)), jnp.sort(jnp.maximum(p0, p1))


@functools.partial(
    pl.kernel,
    out_type=jax.ShapeDtypeStruct((_NTOK // _HALVES,), jnp.float32),
    mesh=plsc.VectorSubcoreMesh(core_axis_name="c", subcore_axis_name="s"),
    compiler_params=pltpu.CompilerParams(needs_layout_passes=False),
    scratch_types=[
        pltpu.VMEM((_CH, _D), jnp.float32),
        pltpu.VMEM((_CH, _D), jnp.float32),
        pltpu.VMEM((_RPW,), jnp.float32),
        pltpu.SemaphoreType.DMA,
        pltpu.SemaphoreType.DMA,
    ],
)
def _sc_thresholds(acts_hbm, thr_hbm, buf_a, buf_b, thrbuf, sem_a, sem_b):
    wid = lax.axis_index("s") * 2 + lax.axis_index("c")
    base = wid * _RPW

    lane = lax.iota(jnp.int32, 16)

    def process(buf, thr_acc, lane_off):
        st = []
        for r in range(_IL):
            a = jnp.sort(buf[r, pl.ds(0, 16)])
            b = jnp.sort(buf[r, pl.ds(16, 16)])
            st.extend(_sorted32(a, b))

        def dbl_step(j, ts):
            nts = []
            for r in range(_IL):
                v1 = jnp.sort(buf[r, pl.ds(32 + j * 32, 16)])
                v2 = jnp.sort(buf[r, pl.ds(48 + j * 32, 16)])
                w0, w1 = _sorted32(v1, v2)
                nts.extend(_top32_merge(ts[2 * r], ts[2 * r + 1], w0, w1))
            return tuple(nts)

        st = lax.fori_loop(0, (_D - 32) // 32, dbl_step, tuple(st))
        for r in range(_IL):
            t = jnp.full((16,), jnp.min(st[2 * r]), jnp.float32)
            thr_acc = jnp.where(lane == lane_off + r, t, thr_acc)
        return thr_acc

    def pair_body(c, carry):
        row0 = base + c * 2 * _CH
        cp_a = pltpu.async_copy(acts_hbm.at[pl.ds(row0, _CH)], buf_a, sem_a)
        cp_b = pltpu.async_copy(acts_hbm.at[pl.ds(row0 + _CH, _CH)], buf_b, sem_b)
        thr_acc = jnp.zeros((16,), jnp.float32)
        cp_a.wait()
        thr_acc = process(buf_a, thr_acc, 0)
        cp_b.wait()
        thr_acc = process(buf_b, thr_acc, _CH)
        thrbuf[pl.ds(c * 2 * _CH, 2 * _CH)] = thr_acc
        return carry

    lax.fori_loop(0, _RPW // (2 * _CH), pair_body, 0)
    pltpu.sync_copy(thrbuf, thr_hbm.at[pl.ds(base, _RPW)])


def _enc_call(x, w_enc, pb, lb):
    n, d = x.shape
    return pl.pallas_call(
        _enc_body,
        grid=(n // _BLK,),
        in_specs=[
            pl.BlockSpec((_BLK, d), lambda i: (i, 0)),
            pl.BlockSpec((d, d), lambda i: (0, 0)),
            pl.BlockSpec((1, d), lambda i: (0, 0)),
            pl.BlockSpec((1, d), lambda i: (0, 0)),
        ],
        out_specs=pl.BlockSpec((_BLK, d), lambda i: (i, 0)),
        out_shape=jax.ShapeDtypeStruct((n, d), jnp.float32),
    )(x, w_enc, pb, lb)


def _dec_call(acts, thr, wd16, pb):
    n, d = acts.shape
    return pl.pallas_call(
        _dec_body,
        grid=(n // _BLK,),
        in_specs=[
            pl.BlockSpec((_BLK, d), lambda i: (i, 0)),
            pl.BlockSpec((_BLK, 1), lambda i: (i, 0)),
            pl.BlockSpec((d, d), lambda i: (0, 0)),
            pl.BlockSpec((1, d), lambda i: (0, 0)),
        ],
        out_specs=pl.BlockSpec((_BLK, d), lambda i: (i, 0)),
        out_shape=jax.ShapeDtypeStruct((n, d), jnp.float32),
    )(acts, thr.reshape(n, 1), wd16, pb)


@jax.jit
def _run(x, w_enc, w_dec, pre_bias, latent_bias):
    n_tok, d = x.shape
    pb = pre_bias.reshape(1, d)
    lb = latent_bias.reshape(1, d)
    wd16 = w_dec.astype(jnp.bfloat16)
    h = n_tok // _HALVES

    acts = [_enc_call(x[i * h:(i + 1) * h], w_enc, pb, lb)
            for i in range(_HALVES)]
    thrs = [_sc_thresholds(a) for a in acts]
    outs = [_dec_call(a, t, wd16, pb) for a, t in zip(acts, thrs)]
    return jnp.concatenate(outs, axis=0)


def kernel(x, ema_frequency_counter, W_enc, W_dec, pre_bias, latent_bias):
    del ema_frequency_counter
    return _run(x, W_enc, W_dec, pre_bias, latent_bias)


# speedup vs baseline: 1.3021x; 1.3021x over previous
import functools

import jax
import jax.numpy as jnp
from jax import lax
from jax.experimental import pallas as pl
from jax.experimental.pallas import tpu as pltpu
from jax.experimental.pallas import tpu_sc as plsc

_K = 32
_D = 768
_BLK = 512
_NTOK = 16384
_HALVES = 1
_NW = 32
_RPW = _NTOK // _HALVES // _NW
_CH = 8
_IL = 8


def _enc_body(x_ref, we_ref, pb_ref, lb_ref, o_ref):
    xc = x_ref[...] - pb_ref[...]
    acts = jnp.dot(xc, we_ref[...], preferred_element_type=jnp.float32)
    o_ref[...] = acts + lb_ref[...]


def _dec_body(a_ref, t_ref, wd_ref, pb_ref, o_ref):
    acts = a_ref[...]
    keep = acts >= t_ref[...]
    acts_topk = jnp.where(keep, jnp.maximum(acts, 0.0), 0.0)
    out = jnp.dot(acts_topk.astype(jnp.bfloat16), wd_ref[...],
                  preferred_element_type=jnp.float32)
    o_ref[...] = out + pb_ref[...]


def _rev(x):
    return lax.rev(x, (0,))


def _sorted32(a, b):
    rb = _rev(b)
    return jnp.sort(jnp.minimum(a, rb)), jnp.sort(jnp.maximum(a, rb))


def _top32_merge(t0, t1, w0, w1):
    p0 = jnp.maximum(t0, _rev(w1))
    p1 = jnp.maximum(t1, _rev(w0))
    return jnp.sort(jnp.minimum(p0, p1)), jnp.sort(jnp.maximum(p0, p1))


@functools.partial(
    pl.kernel,
    out_type=jax.ShapeDtypeStruct((_NTOK // _HALVES,), jnp.float32),
    mesh=plsc.VectorSubcoreMesh(core_axis_name="c", subcore_axis_name="s"),
    compiler_params=pltpu.CompilerParams(needs_layout_passes=False),
    scratch_types=[
        pltpu.VMEM((_CH, _D), jnp.float32),
        pltpu.VMEM((_CH, _D), jnp.float32),
        pltpu.VMEM((_RPW,), jnp.float32),
        pltpu.SemaphoreType.DMA,
        pltpu.SemaphoreType.DMA,
    ],
)
def _sc_thresholds(acts_hbm, thr_hbm, buf_a, buf_b, thrbuf, sem_a, sem_b):
    wid = lax.axis_index("s") * 2 + lax.axis_index("c")
    base = wid * _RPW

    lane = lax.iota(jnp.int32, 16)

    def process(buf, thr_acc, lane_off):
        st = []
        for r in range(_IL):
            a = jnp.sort(buf[r, pl.ds(0, 16)])
            b = jnp.sort(buf[r, pl.ds(16, 16)])
            st.extend(_sorted32(a, b))

        def dbl_step(j, ts):
            nts = []
            for r in range(_IL):
                v1 = jnp.sort(buf[r, pl.ds(32 + j * 32, 16)])
                v2 = jnp.sort(buf[r, pl.ds(48 + j * 32, 16)])
                w0, w1 = _sorted32(v1, v2)
                nts.extend(_top32_merge(ts[2 * r], ts[2 * r + 1], w0, w1))
            return tuple(nts)

        st = lax.fori_loop(0, (_D - 32) // 32, dbl_step, tuple(st), unroll=2)
        for r in range(_IL):
            t = jnp.full((16,), jnp.min(st[2 * r]), jnp.float32)
            thr_acc = jnp.where(lane == lane_off + r, t, thr_acc)
        return thr_acc

    def pair_body(c, carry):
        row0 = base + c * 2 * _CH
        cp_a = pltpu.async_copy(acts_hbm.at[pl.ds(row0, _CH)], buf_a, sem_a)
        cp_b = pltpu.async_copy(acts_hbm.at[pl.ds(row0 + _CH, _CH)], buf_b, sem_b)
        thr_acc = jnp.zeros((16,), jnp.float32)
        cp_a.wait()
        thr_acc = process(buf_a, thr_acc, 0)
        cp_b.wait()
        thr_acc = process(buf_b, thr_acc, _CH)
        thrbuf[pl.ds(c * 2 * _CH, 2 * _CH)] = thr_acc
        return carry

    lax.fori_loop(0, _RPW // (2 * _CH), pair_body, 0)
    pltpu.sync_copy(thrbuf, thr_hbm.at[pl.ds(base, _RPW)])


def _enc_call(x, w_enc, pb, lb):
    n, d = x.shape
    return pl.pallas_call(
        _enc_body,
        grid=(n // _BLK,),
        in_specs=[
            pl.BlockSpec((_BLK, d), lambda i: (i, 0)),
            pl.BlockSpec((d, d), lambda i: (0, 0)),
            pl.BlockSpec((1, d), lambda i: (0, 0)),
            pl.BlockSpec((1, d), lambda i: (0, 0)),
        ],
        out_specs=pl.BlockSpec((_BLK, d), lambda i: (i, 0)),
        out_shape=jax.ShapeDtypeStruct((n, d), jnp.float32),
    )(x, w_enc, pb, lb)


def _dec_call(acts, thr, wd16, pb):
    n, d = acts.shape
    return pl.pallas_call(
        _dec_body,
        grid=(n // _BLK,),
        in_specs=[
            pl.BlockSpec((_BLK, d), lambda i: (i, 0)),
            pl.BlockSpec((_BLK, 1), lambda i: (i, 0)),
            pl.BlockSpec((d, d), lambda i: (0, 0)),
            pl.BlockSpec((1, d), lambda i: (0, 0)),
        ],
        out_specs=pl.BlockSpec((_BLK, d), lambda i: (i, 0)),
        out_shape=jax.ShapeDtypeStruct((n, d), jnp.float32),
    )(acts, thr.reshape(n, 1), wd16, pb)


@jax.jit
def _run(x, w_enc, w_dec, pre_bias, latent_bias):
    n_tok, d = x.shape
    pb = pre_bias.reshape(1, d)
    lb = latent_bias.reshape(1, d)
    wd16 = w_dec.astype(jnp.bfloat16)
    h = n_tok // _HALVES

    acts = [_enc_call(x[i * h:(i + 1) * h], w_enc, pb, lb)
            for i in range(_HALVES)]
    thrs = [_sc_thresholds(a) for a in acts]
    outs = [_dec_call(a, t, wd16, pb) for a, t in zip(acts, thrs)]
    return jnp.concatenate(outs, axis=0)


def kernel(x, ema_frequency_counter, W_enc, W_dec, pre_bias, latent_bias):
    del ema_frequency_counter
    return _run(x, W_enc, W_dec, pre_bias, latent_bias)


# speedup vs baseline: 1.3171x; 1.0115x over previous
import functools

import jax
import jax.numpy as jnp
from jax import lax
from jax.experimental import pallas as pl
from jax.experimental.pallas import tpu as pltpu
from jax.experimental.pallas import tpu_sc as plsc

_K = 32
_D = 768
_BLK = 512
_NTOK = 16384
_HALVES = 1
_NW = 32
_RPW = _NTOK // _HALVES // _NW
_CH = 8
_IL = 8


def _enc_body(x_ref, we_ref, pb_ref, lb_ref, o_ref):
    xc = x_ref[...] - pb_ref[...]
    acts = jnp.dot(xc, we_ref[...], preferred_element_type=jnp.float32)
    o_ref[...] = acts + lb_ref[...]


def _dec_body(a_ref, t_ref, wd_ref, pb_ref, o_ref):
    acts = a_ref[...]
    keep = acts >= t_ref[...]
    acts_topk = jnp.where(keep, jnp.maximum(acts, 0.0), 0.0)
    out = jnp.dot(acts_topk.astype(jnp.bfloat16), wd_ref[...],
                  preferred_element_type=jnp.float32)
    o_ref[...] = out + pb_ref[...]


def _sd(x):
    return plsc.sort_key_val(x, x, descending=True)[0]


def _halves(v1, v2):
    a = jnp.sort(v1)
    bd = _sd(v2)
    return jnp.minimum(a, bd), jnp.maximum(a, bd)


def _top32_merge(t0, t1, w0d, w1d):
    p0 = jnp.maximum(t0, w1d)
    p1 = jnp.maximum(t1, w0d)
    return jnp.sort(jnp.minimum(p0, p1)), jnp.sort(jnp.maximum(p0, p1))


@functools.partial(
    pl.kernel,
    out_type=jax.ShapeDtypeStruct((_NTOK // _HALVES,), jnp.float32),
    mesh=plsc.VectorSubcoreMesh(core_axis_name="c", subcore_axis_name="s"),
    compiler_params=pltpu.CompilerParams(needs_layout_passes=False),
    scratch_types=[
        pltpu.VMEM((_CH, _D), jnp.float32),
        pltpu.VMEM((_CH, _D), jnp.float32),
        pltpu.VMEM((_RPW,), jnp.float32),
        pltpu.SemaphoreType.DMA,
        pltpu.SemaphoreType.DMA,
    ],
)
def _sc_thresholds(acts_hbm, thr_hbm, buf_a, buf_b, thrbuf, sem_a, sem_b):
    wid = lax.axis_index("s") * 2 + lax.axis_index("c")
    base = wid * _RPW

    lane = lax.iota(jnp.int32, 16)

    def process(buf, thr_acc, lane_off):
        st = []
        for r in range(_IL):
            lo, hi = _halves(buf[r, pl.ds(0, 16)], buf[r, pl.ds(16, 16)])
            st.extend((jnp.sort(lo), jnp.sort(hi)))

        def dbl_step(j, ts):
            nts = []
            for r in range(_IL):
                lo, hi = _halves(buf[r, pl.ds(32 + j * 32, 16)],
                                 buf[r, pl.ds(48 + j * 32, 16)])
                nts.extend(_top32_merge(ts[2 * r], ts[2 * r + 1],
                                        _sd(lo), _sd(hi)))
            return tuple(nts)

        st = lax.fori_loop(0, (_D - 32) // 32, dbl_step, tuple(st), unroll=2)
        for r in range(_IL):
            t = jnp.full((16,), jnp.min(st[2 * r]), jnp.float32)
            thr_acc = jnp.where(lane == lane_off + r, t, thr_acc)
        return thr_acc

    def pair_body(c, carry):
        row0 = base + c * 2 * _CH
        cp_a = pltpu.async_copy(acts_hbm.at[pl.ds(row0, _CH)], buf_a, sem_a)
        cp_b = pltpu.async_copy(acts_hbm.at[pl.ds(row0 + _CH, _CH)], buf_b, sem_b)
        thr_acc = jnp.zeros((16,), jnp.float32)
        cp_a.wait()
        thr_acc = process(buf_a, thr_acc, 0)
        cp_b.wait()
        thr_acc = process(buf_b, thr_acc, _CH)
        thrbuf[pl.ds(c * 2 * _CH, 2 * _CH)] = thr_acc
        return carry

    lax.fori_loop(0, _RPW // (2 * _CH), pair_body, 0)
    pltpu.sync_copy(thrbuf, thr_hbm.at[pl.ds(base, _RPW)])


def _enc_call(x, w_enc, pb, lb):
    n, d = x.shape
    return pl.pallas_call(
        _enc_body,
        grid=(n // _BLK,),
        in_specs=[
            pl.BlockSpec((_BLK, d), lambda i: (i, 0)),
            pl.BlockSpec((d, d), lambda i: (0, 0)),
            pl.BlockSpec((1, d), lambda i: (0, 0)),
            pl.BlockSpec((1, d), lambda i: (0, 0)),
        ],
        out_specs=pl.BlockSpec((_BLK, d), lambda i: (i, 0)),
        out_shape=jax.ShapeDtypeStruct((n, d), jnp.float32),
    )(x, w_enc, pb, lb)


def _dec_call(acts, thr, wd16, pb):
    n, d = acts.shape
    return pl.pallas_call(
        _dec_body,
        grid=(n // _BLK,),
        in_specs=[
            pl.BlockSpec((_BLK, d), lambda i: (i, 0)),
            pl.BlockSpec((_BLK, 1), lambda i: (i, 0)),
            pl.BlockSpec((d, d), lambda i: (0, 0)),
            pl.BlockSpec((1, d), lambda i: (0, 0)),
        ],
        out_specs=pl.BlockSpec((_BLK, d), lambda i: (i, 0)),
        out_shape=jax.ShapeDtypeStruct((n, d), jnp.float32),
    )(acts, thr.reshape(n, 1), wd16, pb)


@jax.jit
def _run(x, w_enc, w_dec, pre_bias, latent_bias):
    n_tok, d = x.shape
    pb = pre_bias.reshape(1, d)
    lb = latent_bias.reshape(1, d)
    wd16 = w_dec.astype(jnp.bfloat16)
    h = n_tok // _HALVES

    acts = [_enc_call(x[i * h:(i + 1) * h], w_enc, pb, lb)
            for i in range(_HALVES)]
    thrs = [_sc_thresholds(a) for a in acts]
    outs = [_dec_call(a, t, wd16, pb) for a, t in zip(acts, thrs)]
    return jnp.concatenate(outs, axis=0)


def kernel(x, ema_frequency_counter, W_enc, W_dec, pre_bias, latent_bias):
    del ema_frequency_counter
    return _run(x, W_enc, W_dec, pre_bias, latent_bias)


# speedup vs baseline: 1.4308x; 1.0863x over previous
import functools

import jax
import jax.numpy as jnp
from jax import lax
from jax.experimental import pallas as pl
from jax.experimental.pallas import tpu as pltpu
from jax.experimental.pallas import tpu_sc as plsc

_K = 32
_D = 768
_BLK = 512
_NTOK = 16384
_HALVES = 1
_NW = 32
_RPW = _NTOK // _HALVES // _NW
_CH = 16
_NBUF = 4
_IL = 8


def _enc_body(x_ref, we_ref, pb_ref, lb_ref, o_ref):
    xc = x_ref[...] - pb_ref[...]
    acts = jnp.dot(xc, we_ref[...], preferred_element_type=jnp.float32)
    o_ref[...] = acts + lb_ref[...]


def _dec_body(a_ref, t_ref, wd_ref, pb_ref, o_ref):
    acts = a_ref[...]
    keep = acts >= t_ref[...]
    acts_topk = jnp.where(keep, jnp.maximum(acts, 0.0), 0.0)
    out = jnp.dot(acts_topk.astype(jnp.bfloat16), wd_ref[...],
                  preferred_element_type=jnp.float32)
    o_ref[...] = out + pb_ref[...]


def _sd(x):
    return plsc.sort_key_val(x, x, descending=True)[0]


def _halves(v1, v2):
    a = jnp.sort(v1)
    bd = _sd(v2)
    return jnp.minimum(a, bd), jnp.maximum(a, bd)


def _top32_merge(t0, t1, w0d, w1d):
    p0 = jnp.maximum(t0, w1d)
    p1 = jnp.maximum(t1, w0d)
    return jnp.sort(jnp.minimum(p0, p1)), jnp.sort(jnp.maximum(p0, p1))


@functools.partial(
    pl.kernel,
    out_type=jax.ShapeDtypeStruct((_NTOK // _HALVES,), jnp.float32),
    mesh=plsc.VectorSubcoreMesh(core_axis_name="c", subcore_axis_name="s"),
    compiler_params=pltpu.CompilerParams(needs_layout_passes=False),
    scratch_types=(
        [pltpu.VMEM((_CH, _D), jnp.float32)] * _NBUF
        + [pltpu.VMEM((_RPW,), jnp.float32)]
        + [pltpu.SemaphoreType.DMA] * _NBUF
    ),
)
def _sc_thresholds(acts_hbm, thr_hbm, b0, b1, b2, b3, thrbuf, s0, s1, s2, s3):
    bufs = (b0, b1, b2, b3)
    sems = (s0, s1, s2, s3)
    wid = lax.axis_index("s") * 2 + lax.axis_index("c")
    base = wid * _RPW
    nchunks = _RPW // _CH

    lane = lax.iota(jnp.int32, 16)

    def process(buf, thr_acc, lane_off):
        st = []
        for r in range(_IL):
            rr = lane_off + r
            lo, hi = _halves(buf[rr, pl.ds(0, 16)], buf[rr, pl.ds(16, 16)])
            st.extend((jnp.sort(lo), jnp.sort(hi)))

        def dbl_step(j, ts):
            nts = []
            for r in range(_IL):
                rr = lane_off + r
                lo, hi = _halves(buf[rr, pl.ds(32 + j * 32, 16)],
                                 buf[rr, pl.ds(48 + j * 32, 16)])
                nts.extend(_top32_merge(ts[2 * r], ts[2 * r + 1],
                                        _sd(lo), _sd(hi)))
            return tuple(nts)

        st = lax.fori_loop(0, (_D - 32) // 32, dbl_step, tuple(st), unroll=2)
        for r in range(_IL):
            t = jnp.full((16,), jnp.min(st[2 * r]), jnp.float32)
            thr_acc = jnp.where(lane == lane_off + r, t, thr_acc)
        return thr_acc

    for q in range(_NBUF):
        pltpu.async_copy(acts_hbm.at[pl.ds(base + q * _CH, _CH)],
                         bufs[q], sems[q])

    def super_body(cc, carry):
        for q in range(_NBUF):
            chunk = cc * _NBUF + q
            pltpu.make_async_copy(
                acts_hbm.at[pl.ds(base + chunk * _CH, _CH)],
                bufs[q], sems[q]).wait()
            thr_acc = jnp.zeros((16,), jnp.float32)
            thr_acc = process(bufs[q], thr_acc, 0)
            thr_acc = process(bufs[q], thr_acc, _IL)
            thrbuf[pl.ds(chunk * _CH, _CH)] = thr_acc

            nxt = chunk + _NBUF

            @pl.when(nxt < nchunks)
            def _prefetch():
                pltpu.async_copy(acts_hbm.at[pl.ds(base + nxt * _CH, _CH)],
                                 bufs[q], sems[q])
        return carry

    lax.fori_loop(0, nchunks // _NBUF, super_body, 0)
    pltpu.sync_copy(thrbuf, thr_hbm.at[pl.ds(base, _RPW)])


def _enc_call(x, w_enc, pb, lb):
    n, d = x.shape
    return pl.pallas_call(
        _enc_body,
        grid=(n // _BLK,),
        in_specs=[
            pl.BlockSpec((_BLK, d), lambda i: (i, 0)),
            pl.BlockSpec((d, d), lambda i: (0, 0)),
            pl.BlockSpec((1, d), lambda i: (0, 0)),
            pl.BlockSpec((1, d), lambda i: (0, 0)),
        ],
        out_specs=pl.BlockSpec((_BLK, d), lambda i: (i, 0)),
        out_shape=jax.ShapeDtypeStruct((n, d), jnp.float32),
    )(x, w_enc, pb, lb)


def _dec_call(acts, thr, wd16, pb):
    n, d = acts.shape
    return pl.pallas_call(
        _dec_body,
        grid=(n // _BLK,),
        in_specs=[
            pl.BlockSpec((_BLK, d), lambda i: (i, 0)),
            pl.BlockSpec((_BLK, 1), lambda i: (i, 0)),
            pl.BlockSpec((d, d), lambda i: (0, 0)),
            pl.BlockSpec((1, d), lambda i: (0, 0)),
        ],
        out_specs=pl.BlockSpec((_BLK, d), lambda i: (i, 0)),
        out_shape=jax.ShapeDtypeStruct((n, d), jnp.float32),
    )(acts, thr.reshape(n, 1), wd16, pb)


@jax.jit
def _run(x, w_enc, w_dec, pre_bias, latent_bias):
    n_tok, d = x.shape
    pb = pre_bias.reshape(1, d)
    lb = latent_bias.reshape(1, d)
    wd16 = w_dec.astype(jnp.bfloat16)
    h = n_tok // _HALVES

    acts = [_enc_call(x[i * h:(i + 1) * h], w_enc, pb, lb)
            for i in range(_HALVES)]
    thrs = [_sc_thresholds(a) for a in acts]
    outs = [_dec_call(a, t, wd16, pb) for a, t in zip(acts, thrs)]
    return jnp.concatenate(outs, axis=0)


def kernel(x, ema_frequency_counter, W_enc, W_dec, pre_bias, latent_bias):
    del ema_frequency_counter
    return _run(x, W_enc, W_dec, pre_bias, latent_bias)


# speedup vs baseline: 1.5916x; 1.1124x over previous
import functools

import jax
import jax.numpy as jnp
from jax import lax
from jax.experimental import pallas as pl
from jax.experimental.pallas import tpu as pltpu
from jax.experimental.pallas import tpu_sc as plsc

_K = 32
_D = 768
_BLK = 1024
_NTOK = 16384
_HALVES = 1
_NW = 32
_RPW = _NTOK // _HALVES // _NW
_CH = 16
_NBUF = 4
_IL = 8


def _enc_body(x_ref, we_ref, pb_ref, lb_ref, o_ref):
    xc = x_ref[...] - pb_ref[...]
    acts = jnp.dot(xc, we_ref[...], preferred_element_type=jnp.float32)
    o_ref[...] = acts + lb_ref[...]


def _dec_body(a_ref, t_ref, wd_ref, pb_ref, o_ref):
    acts = a_ref[...]
    keep = acts >= t_ref[...]
    acts_topk = jnp.where(keep, jnp.maximum(acts, 0.0), 0.0)
    out = jnp.dot(acts_topk.astype(jnp.bfloat16), wd_ref[...],
                  preferred_element_type=jnp.float32)
    o_ref[...] = out + pb_ref[...]


def _sd(x):
    return plsc.sort_key_val(x, x, descending=True)[0]


def _halves(v1, v2):
    a = jnp.sort(v1)
    bd = _sd(v2)
    return jnp.minimum(a, bd), jnp.maximum(a, bd)


def _top32_merge(t0, t1, w0d, w1d):
    p0 = jnp.maximum(t0, w1d)
    p1 = jnp.maximum(t1, w0d)
    return jnp.sort(jnp.minimum(p0, p1)), jnp.sort(jnp.maximum(p0, p1))


@functools.partial(
    pl.kernel,
    out_type=jax.ShapeDtypeStruct((_NTOK // _HALVES,), jnp.float32),
    mesh=plsc.VectorSubcoreMesh(core_axis_name="c", subcore_axis_name="s"),
    compiler_params=pltpu.CompilerParams(needs_layout_passes=False),
    scratch_types=(
        [pltpu.VMEM((_CH, _D), jnp.float32)] * _NBUF
        + [pltpu.VMEM((_RPW,), jnp.float32)]
        + [pltpu.SemaphoreType.DMA] * _NBUF
    ),
)
def _sc_thresholds(acts_hbm, thr_hbm, b0, b1, b2, b3, thrbuf, s0, s1, s2, s3):
    bufs = (b0, b1, b2, b3)
    sems = (s0, s1, s2, s3)
    wid = lax.axis_index("s") * 2 + lax.axis_index("c")
    base = wid * _RPW
    nchunks = _RPW // _CH

    lane = lax.iota(jnp.int32, 16)

    def process(buf, thr_acc, lane_off):
        st = []
        for r in range(_IL):
            rr = lane_off + r
            lo, hi = _halves(buf[rr, pl.ds(0, 16)], buf[rr, pl.ds(16, 16)])
            st.extend((jnp.sort(lo), jnp.sort(hi)))

        def dbl_step(j, ts):
            nts = []
            for r in range(_IL):
                rr = lane_off + r
                lo, hi = _halves(buf[rr, pl.ds(32 + j * 32, 16)],
                                 buf[rr, pl.ds(48 + j * 32, 16)])
                nts.extend(_top32_merge(ts[2 * r], ts[2 * r + 1],
                                        _sd(lo), _sd(hi)))
            return tuple(nts)

        st = lax.fori_loop(0, (_D - 32) // 32, dbl_step, tuple(st), unroll=2)
        for r in range(_IL):
            t = jnp.full((16,), jnp.min(st[2 * r]), jnp.float32)
            thr_acc = jnp.where(lane == lane_off + r, t, thr_acc)
        return thr_acc

    for q in range(_NBUF):
        pltpu.async_copy(acts_hbm.at[pl.ds(base + q * _CH, _CH)],
                         bufs[q], sems[q])

    def super_body(cc, carry):
        for q in range(_NBUF):
            chunk = cc * _NBUF + q
            pltpu.make_async_copy(
                acts_hbm.at[pl.ds(base + chunk * _CH, _CH)],
                bufs[q], sems[q]).wait()
            thr_acc = jnp.zeros((16,), jnp.float32)
            thr_acc = process(bufs[q], thr_acc, 0)
            thr_acc = process(bufs[q], thr_acc, _IL)
            thrbuf[pl.ds(chunk * _CH, _CH)] = thr_acc

            nxt = chunk + _NBUF

            @pl.when(nxt < nchunks)
            def _prefetch():
                pltpu.async_copy(acts_hbm.at[pl.ds(base + nxt * _CH, _CH)],
                                 bufs[q], sems[q])
        return carry

    lax.fori_loop(0, nchunks // _NBUF, super_body, 0)
    pltpu.sync_copy(thrbuf, thr_hbm.at[pl.ds(base, _RPW)])


def _enc_call(x, w_enc, pb, lb):
    n, d = x.shape
    return pl.pallas_call(
        _enc_body,
        grid=(n // _BLK,),
        in_specs=[
            pl.BlockSpec((_BLK, d), lambda i: (i, 0)),
            pl.BlockSpec((d, d), lambda i: (0, 0)),
            pl.BlockSpec((1, d), lambda i: (0, 0)),
            pl.BlockSpec((1, d), lambda i: (0, 0)),
        ],
        out_specs=pl.BlockSpec((_BLK, d), lambda i: (i, 0)),
        out_shape=jax.ShapeDtypeStruct((n, d), jnp.float32),
    )(x, w_enc, pb, lb)


def _dec_call(acts, thr, wd16, pb):
    n, d = acts.shape
    return pl.pallas_call(
        _dec_body,
        grid=(n // _BLK,),
        in_specs=[
            pl.BlockSpec((_BLK, d), lambda i: (i, 0)),
            pl.BlockSpec((_BLK, 1), lambda i: (i, 0)),
            pl.BlockSpec((d, d), lambda i: (0, 0)),
            pl.BlockSpec((1, d), lambda i: (0, 0)),
        ],
        out_specs=pl.BlockSpec((_BLK, d), lambda i: (i, 0)),
        out_shape=jax.ShapeDtypeStruct((n, d), jnp.float32),
    )(acts, thr.reshape(n, 1), wd16, pb)


@jax.jit
def _run(x, w_enc, w_dec, pre_bias, latent_bias):
    n_tok, d = x.shape
    pb = pre_bias.reshape(1, d)
    lb = latent_bias.reshape(1, d)
    wd16 = w_dec.astype(jnp.bfloat16)
    h = n_tok // _HALVES

    acts = [_enc_call(x[i * h:(i + 1) * h], w_enc, pb, lb)
            for i in range(_HALVES)]
    thrs = [_sc_thresholds(a) for a in acts]
    outs = [_dec_call(a, t, wd16, pb) for a, t in zip(acts, thrs)]
    return jnp.concatenate(outs, axis=0)


def kernel(x, ema_frequency_counter, W_enc, W_dec, pre_bias, latent_bias):
    del ema_frequency_counter
    return _run(x, W_enc, W_dec, pre_bias, latent_bias)


# speedup vs baseline: 1.6602x; 1.0431x over previous
import functools

import jax
import jax.numpy as jnp
from jax import lax
from jax.experimental import pallas as pl
from jax.experimental.pallas import tpu as pltpu
from jax.experimental.pallas import tpu_sc as plsc

_K = 32
_D = 768
_BLK = 2048
_NTOK = 16384
_HALVES = 1
_NW = 32
_RPW = _NTOK // _HALVES // _NW
_CH = 16
_NBUF = 4
_IL = 8


def _enc_body(x_ref, we_ref, pb_ref, lb_ref, o_ref):
    xc = x_ref[...] - pb_ref[...]
    acts = jnp.dot(xc, we_ref[...], preferred_element_type=jnp.float32)
    o_ref[...] = acts + lb_ref[...]


def _dec_body(a_ref, t_ref, wd_ref, pb_ref, o_ref):
    acts = a_ref[...]
    keep = acts >= t_ref[...]
    acts_topk = jnp.where(keep, jnp.maximum(acts, 0.0), 0.0)
    out = jnp.dot(acts_topk.astype(jnp.bfloat16), wd_ref[...],
                  preferred_element_type=jnp.float32)
    o_ref[...] = out + pb_ref[...]


def _sd(x):
    return plsc.sort_key_val(x, x, descending=True)[0]


def _halves(v1, v2):
    a = jnp.sort(v1)
    bd = _sd(v2)
    return jnp.minimum(a, bd), jnp.maximum(a, bd)


def _top32_merge(t0, t1, w0d, w1d):
    p0 = jnp.maximum(t0, w1d)
    p1 = jnp.maximum(t1, w0d)
    return jnp.sort(jnp.minimum(p0, p1)), jnp.sort(jnp.maximum(p0, p1))


@functools.partial(
    pl.kernel,
    out_type=jax.ShapeDtypeStruct((_NTOK // _HALVES,), jnp.float32),
    mesh=plsc.VectorSubcoreMesh(core_axis_name="c", subcore_axis_name="s"),
    compiler_params=pltpu.CompilerParams(needs_layout_passes=False),
    scratch_types=(
        [pltpu.VMEM((_CH, _D), jnp.float32)] * _NBUF
        + [pltpu.VMEM((_RPW,), jnp.float32)]
        + [pltpu.SemaphoreType.DMA] * _NBUF
    ),
)
def _sc_thresholds(acts_hbm, thr_hbm, b0, b1, b2, b3, thrbuf, s0, s1, s2, s3):
    bufs = (b0, b1, b2, b3)
    sems = (s0, s1, s2, s3)
    wid = lax.axis_index("s") * 2 + lax.axis_index("c")
    base = wid * _RPW
    nchunks = _RPW // _CH

    lane = lax.iota(jnp.int32, 16)

    def process(buf, thr_acc, lane_off):
        st = []
        for r in range(_IL):
            rr = lane_off + r
            lo, hi = _halves(buf[rr, pl.ds(0, 16)], buf[rr, pl.ds(16, 16)])
            st.extend((jnp.sort(lo), jnp.sort(hi)))

        def dbl_step(j, ts):
            nts = []
            for r in range(_IL):
                rr = lane_off + r
                lo, hi = _halves(buf[rr, pl.ds(32 + j * 32, 16)],
                                 buf[rr, pl.ds(48 + j * 32, 16)])
                nts.extend(_top32_merge(ts[2 * r], ts[2 * r + 1],
                                        _sd(lo), _sd(hi)))
            return tuple(nts)

        st = lax.fori_loop(0, (_D - 32) // 32, dbl_step, tuple(st), unroll=2)
        for r in range(_IL):
            t = jnp.full((16,), jnp.min(st[2 * r]), jnp.float32)
            thr_acc = jnp.where(lane == lane_off + r, t, thr_acc)
        return thr_acc

    for q in range(_NBUF):
        pltpu.async_copy(acts_hbm.at[pl.ds(base + q * _CH, _CH)],
                         bufs[q], sems[q])

    def super_body(cc, carry):
        for q in range(_NBUF):
            chunk = cc * _NBUF + q
            pltpu.make_async_copy(
                acts_hbm.at[pl.ds(base + chunk * _CH, _CH)],
                bufs[q], sems[q]).wait()
            thr_acc = jnp.zeros((16,), jnp.float32)
            thr_acc = process(bufs[q], thr_acc, 0)
            thr_acc = process(bufs[q], thr_acc, _IL)
            thrbuf[pl.ds(chunk * _CH, _CH)] = thr_acc

            nxt = chunk + _NBUF

            @pl.when(nxt < nchunks)
            def _prefetch():
                pltpu.async_copy(acts_hbm.at[pl.ds(base + nxt * _CH, _CH)],
                                 bufs[q], sems[q])
        return carry

    lax.fori_loop(0, nchunks // _NBUF, super_body, 0)
    pltpu.sync_copy(thrbuf, thr_hbm.at[pl.ds(base, _RPW)])


def _enc_call(x, w_enc, pb, lb):
    n, d = x.shape
    return pl.pallas_call(
        _enc_body,
        grid=(n // _BLK,),
        in_specs=[
            pl.BlockSpec((_BLK, d), lambda i: (i, 0)),
            pl.BlockSpec((d, d), lambda i: (0, 0)),
            pl.BlockSpec((1, d), lambda i: (0, 0)),
            pl.BlockSpec((1, d), lambda i: (0, 0)),
        ],
        out_specs=pl.BlockSpec((_BLK, d), lambda i: (i, 0)),
        out_shape=jax.ShapeDtypeStruct((n, d), jnp.float32),
    )(x, w_enc, pb, lb)


def _dec_call(acts, thr, wd16, pb):
    n, d = acts.shape
    return pl.pallas_call(
        _dec_body,
        grid=(n // _BLK,),
        in_specs=[
            pl.BlockSpec((_BLK, d), lambda i: (i, 0)),
            pl.BlockSpec((_BLK, 1), lambda i: (i, 0)),
            pl.BlockSpec((d, d), lambda i: (0, 0)),
            pl.BlockSpec((1, d), lambda i: (0, 0)),
        ],
        out_specs=pl.BlockSpec((_BLK, d), lambda i: (i, 0)),
        out_shape=jax.ShapeDtypeStruct((n, d), jnp.float32),
    )(acts, thr.reshape(n, 1), wd16, pb)


@jax.jit
def _run(x, w_enc, w_dec, pre_bias, latent_bias):
    n_tok, d = x.shape
    pb = pre_bias.reshape(1, d)
    lb = latent_bias.reshape(1, d)
    wd16 = w_dec.astype(jnp.bfloat16)
    h = n_tok // _HALVES

    acts = [_enc_call(x[i * h:(i + 1) * h], w_enc, pb, lb)
            for i in range(_HALVES)]
    thrs = [_sc_thresholds(a) for a in acts]
    outs = [_dec_call(a, t, wd16, pb) for a, t in zip(acts, thrs)]
    return jnp.concatenate(outs, axis=0)


def kernel(x, ema_frequency_counter, W_enc, W_dec, pre_bias, latent_bias):
    del ema_frequency_counter
    return _run(x, W_enc, W_dec, pre_bias, latent_bias)
